# baseline (speedup 1.0000x reference)
import functools

import jax
import jax.numpy as jnp
from jax import lax
from jax.experimental import pallas as pl
from jax.experimental.pallas import tpu as pltpu
from jax.experimental.pallas import tpu_sc as plsc

N = 10000
E = 320000
D_IN = 128
D_H = 64
NG = 64

NC = 2
NS = 16
L = 16
NW = NC * NS
CHUNK = 128
NCHUNK = 80
EPW = E // NW
FULL = EPW // CHUNK
TAIL = EPW - FULL * CHUNK
N_PAD = 10240
RPT = N_PAD // NS
RING = 8
BLK = 1024

_mesh = plsc.VectorSubcoreMesh(core_axis_name="c", subcore_axis_name="s")
_sc_params = pltpu.CompilerParams(use_tc_tiling_on_sc=False)


def _pad_vec(wid, k):
    i16 = lax.iota(jnp.int32, L)
    return N + lax.rem(i16 + L * k + wid * 7, N_PAD - N)


def _load_slab(flat_hbm, idx_v, sem, wid):
    base = wid * EPW

    def enq(j, carry):
        pltpu.async_copy(flat_hbm.at[pl.ds(base + j * CHUNK, CHUNK)],
                         idx_v.at[j], sem)
        return carry

    lax.fori_loop(0, FULL, enq, 0)
    pltpu.sync_copy(flat_hbm.at[pl.ds(base + FULL * CHUNK, TAIL)],
                    idx_v.at[FULL, pl.ds(0, TAIL)])
    for k in range(TAIL // L, CHUNK // L):
        idx_v[FULL, pl.ds(k * L, L)] = _pad_vec(wid, k)
    for k in range(CHUNK // L):
        idx_v[NCHUNK - 1, pl.ds(k * L, L)] = _pad_vec(wid, 8 + k)

    def drain(j, carry):
        pltpu.make_async_copy(flat_hbm.at[pl.ds(base + j * CHUNK, CHUNK)],
                              idx_v.at[j], sem).wait()
        return carry

    lax.fori_loop(0, FULL, drain, 0)


@functools.partial(
    pl.kernel,
    out_type=jax.ShapeDtypeStruct((NC, N_PAD), jnp.float32),
    mesh=_mesh,
    scratch_types=[
        pltpu.VMEM_SHARED((N_PAD,), jnp.float32),
        pltpu.VMEM((NCHUNK, CHUNK), jnp.int32),
        pltpu.VMEM((CHUNK,), jnp.float32),
        pltpu.VMEM((RPT,), jnp.float32),
        pltpu.SemaphoreType.DMA,
    ],
    compiler_params=_sc_params,
)
def _sc_degree(dst_hbm, deg_out, deg_sh, idx_v, ones_v, zero_v, lsem):
    c = lax.axis_index("c")
    s = lax.axis_index("s")
    wid = c * NS + s
    for i in range(CHUNK // L):
        ones_v[pl.ds(i * L, L)] = jnp.full((L,), 1.0, jnp.float32)
    for i in range(RPT // L):
        zero_v[pl.ds(i * L, L)] = jnp.zeros((L,), jnp.float32)
    pltpu.sync_copy(zero_v, deg_sh.at[pl.ds(s * RPT, RPT)])
    _load_slab(dst_hbm, idx_v, lsem, wid)
    plsc.subcore_barrier()

    def body(j, carry):
        pltpu.sync_copy(ones_v, deg_sh.at[idx_v.at[j]], add=True)
        return carry

    lax.fori_loop(0, NCHUNK, body, 0)
    plsc.subcore_barrier()
    pltpu.sync_copy(deg_sh.at[pl.ds(s * RPT, RPT)],
                    deg_out.at[c, pl.ds(s * RPT, RPT)])


@functools.partial(
    pl.kernel,
    out_type=jax.ShapeDtypeStruct((N_PAD, NC * D_H), jnp.float32),
    mesh=_mesh,
    scratch_types=[
        pltpu.VMEM_SHARED((N_PAD, D_H), jnp.float32),
        pltpu.VMEM((NCHUNK, CHUNK), jnp.int32),
        pltpu.VMEM((NCHUNK, CHUNK), jnp.int32),
        pltpu.VMEM((RING, CHUNK, D_H), jnp.float32),
        pltpu.VMEM((64, D_H), jnp.float32),
        pltpu.SemaphoreType.DMA((RING,)),
        pltpu.SemaphoreType.DMA((RING,)),
        pltpu.SemaphoreType.DMA,
    ],
    compiler_params=_sc_params,
)
def _sc_scatter(g_hbm, src_hbm, dst_hbm, acc_out,
                acc_sh, src_v, dst_v, rows, zero_v, sem_g, sem_s, lsem):
    c = lax.axis_index("c")
    s = lax.axis_index("s")
    wid = c * NS + s
    for i in range(64):
        for k in range(D_H // L):
            zero_v[i, pl.ds(k * L, L)] = jnp.zeros((L,), jnp.float32)
    for k in range(RPT // 64):
        pltpu.sync_copy(zero_v, acc_sh.at[pl.ds(s * RPT + k * 64, 64)])
    _load_slab(src_hbm, src_v, lsem, wid)
    _load_slab(dst_hbm, dst_v, lsem, wid)
    plsc.subcore_barrier()
    for k in range(RING):
        pltpu.async_copy(g_hbm.at[src_v.at[k]], rows.at[k], sem_g.at[k])

    def body(t, carry):
        base = RING * t
        for k in range(RING):
            pltpu.make_async_copy(g_hbm.at[src_v.at[base + k]],
                                  rows.at[k], sem_g.at[k]).wait()
            pltpu.async_copy(rows.at[k], acc_sh.at[dst_v.at[base + k]],
                             sem_s.at[k], add=True)
        for k in range(RING):
            pltpu.make_async_copy(rows.at[k], acc_sh.at[dst_v.at[base + k]],
                                  sem_s.at[k]).wait()

            @pl.when(base + k + RING < NCHUNK)
            def _():
                pltpu.async_copy(g_hbm.at[src_v.at[base + k + RING]],
                                 rows.at[k], sem_g.at[k])

        return carry

    lax.fori_loop(0, NCHUNK // RING, body, 0)
    plsc.subcore_barrier()
    pltpu.sync_copy(acc_sh.at[pl.ds(s * RPT, RPT)],
                    acc_out.at[pl.ds(s * RPT, RPT), pl.ds(c * D_H, D_H)])


def _dinv_col(deg_ref):
    dv = lax.rsqrt(deg_ref[0:1, :] + deg_ref[1:2, :] + 1.0)
    return jnp.transpose(dv, (1, 0))


def _mm_scale_body(x_ref, w_ref, deg_ref, g_ref):
    g_ref[...] = jnp.dot(x_ref[...] * _dinv_col(deg_ref), w_ref[...],
                         preferred_element_type=jnp.float32)


def _mm_scale(x, w, deg_part):
    m, k = x.shape
    n = w.shape[1]
    return pl.pallas_call(
        _mm_scale_body,
        grid=(m // BLK,),
        in_specs=[pl.BlockSpec((BLK, k), lambda i: (i, 0)),
                  pl.BlockSpec((k, n), lambda i: (0, 0)),
                  pl.BlockSpec((NC, BLK), lambda i: (0, i))],
        out_specs=pl.BlockSpec((BLK, n), lambda i: (i, 0)),
        out_shape=jax.ShapeDtypeStruct((m, n), jnp.float32),
    )(x, w, deg_part)


def _layer2_body(acc_ref, g_ref, deg_ref, b_ref, w_ref, o_ref):
    dv = _dinv_col(deg_ref)
    accsum = acc_ref[:, :D_H] + acc_ref[:, D_H:]
    h1 = jnp.maximum(dv * (accsum + g_ref[...]) + b_ref[...], 0.0)
    o_ref[...] = jnp.dot(h1, w_ref[...], preferred_element_type=jnp.float32) * dv


def _layer2(acc, g1, deg_part, b1, w2):
    return pl.pallas_call(
        _layer2_body,
        grid=(N_PAD // BLK,),
        in_specs=[pl.BlockSpec((BLK, NC * D_H), lambda i: (i, 0)),
                  pl.BlockSpec((BLK, D_H), lambda i: (i, 0)),
                  pl.BlockSpec((NC, BLK), lambda i: (0, i)),
                  pl.BlockSpec((1, D_H), lambda i: (0, 0)),
                  pl.BlockSpec((D_H, D_H), lambda i: (0, 0))],
        out_specs=pl.BlockSpec((BLK, D_H), lambda i: (i, 0)),
        out_shape=jax.ShapeDtypeStruct((N_PAD, D_H), jnp.float32),
    )(acc, g1, deg_part, b1, w2)


def _pool_body(acc_ref, g_ref, deg_ref, b_ref, batch_ref, wl_ref, bl_ref,
               o_ref, sums_scr, cnt_scr):
    i = pl.program_id(0)

    @pl.when(i == 0)
    def _():
        sums_scr[...] = jnp.zeros_like(sums_scr)
        cnt_scr[...] = jnp.zeros_like(cnt_scr)

    dv = _dinv_col(deg_ref)
    accsum = acc_ref[:, :D_H] + acc_ref[:, D_H:]
    h2 = jnp.maximum(dv * (accsum + g_ref[...]) + b_ref[...], 0.0)
    pt = (batch_ref[...] == lax.broadcasted_iota(jnp.int32, (NG, BLK), 0))
    pt = pt.astype(jnp.float32)
    sums_scr[...] += lax.dot_general(pt, h2, (((1,), (0,)), ((), ())),
                                     preferred_element_type=jnp.float32)
    cnt_scr[...] += lax.dot_general(pt, jnp.ones((BLK, 1), jnp.float32),
                                    (((1,), (0,)), ((), ())),
                                    preferred_element_type=jnp.float32)

    @pl.when(i == pl.num_programs(0) - 1)
    def _():
        pooled = sums_scr[...] / jnp.maximum(cnt_scr[...], 1.0)
        o_ref[...] = jnp.dot(pooled, wl_ref[...],
                             preferred_element_type=jnp.float32) + bl_ref[...]


def _pool(acc, g2, deg_part, b2, batch_row, wlin, blin):
    return pl.pallas_call(
        _pool_body,
        grid=(N_PAD // BLK,),
        in_specs=[pl.BlockSpec((BLK, NC * D_H), lambda i: (i, 0)),
                  pl.BlockSpec((BLK, D_H), lambda i: (i, 0)),
                  pl.BlockSpec((NC, BLK), lambda i: (0, i)),
                  pl.BlockSpec((1, D_H), lambda i: (0, 0)),
                  pl.BlockSpec((1, BLK), lambda i: (0, i)),
                  pl.BlockSpec((D_H, 2), lambda i: (0, 0)),
                  pl.BlockSpec((1, 2), lambda i: (0, 0))],
        out_specs=pl.BlockSpec((NG, 2), lambda i: (0, 0)),
        out_shape=jax.ShapeDtypeStruct((NG, 2), jnp.float32),
        scratch_shapes=[pltpu.VMEM((NG, D_H), jnp.float32),
                        pltpu.VMEM((NG, 1), jnp.float32)],
    )(acc, g2, deg_part, b2, batch_row, wlin, blin)


def kernel(x, edge_index, batch, W1, b1, W2, b2, Wlin, blin):
    src = edge_index[0].astype(jnp.int32)
    dst = edge_index[1].astype(jnp.int32)
    x_pad = jnp.concatenate([x, jnp.zeros((N_PAD - N, D_IN), x.dtype)])
    batch_row = jnp.concatenate(
        [batch.astype(jnp.int32), jnp.full((N_PAD - N,), NG, jnp.int32)]
    ).reshape(1, N_PAD)

    deg_part = _sc_degree(dst)
    g1 = _mm_scale(x_pad, W1, deg_part)
    acc1 = _sc_scatter(g1, src, dst)
    g2 = _layer2(acc1, g1, deg_part, b1.reshape(1, D_H), W2)
    acc2 = _sc_scatter(g2, src, dst)
    return _pool(acc2, g2, deg_part, b2.reshape(1, D_H), batch_row,
                 Wlin, blin.reshape(1, 2))


# speedup vs baseline: 1.0582x; 1.0582x over previous
import functools

import jax
import jax.numpy as jnp
from jax import lax
from jax.experimental import pallas as pl
from jax.experimental.pallas import tpu as pltpu
from jax.experimental.pallas import tpu_sc as plsc

N = 10000
E = 320000
D_IN = 128
D_H = 64
NG = 64

NC = 2
NS = 16
L = 16
NW = NC * NS
CHUNK = 128
NCHUNK = 80
ECHUNK = E // CHUNK
EBOX = 500
EBOXP = 512
NROW = NW * NCHUNK
N_PAD = 10240
RPT = N_PAD // NS
RING = 8
BLK = 1024

_mesh = plsc.VectorSubcoreMesh(core_axis_name="c", subcore_axis_name="s")
_sc_params = pltpu.CompilerParams(use_tc_tiling_on_sc=False)


def _edge_split_body(e_ref, s_ref, d_ref):
    for k in range(EBOX):
        s_ref[pl.ds(k, 1), :] = e_ref[0:1, pl.ds(k * CHUNK, CHUNK)]
        d_ref[pl.ds(k, 1), :] = e_ref[1:2, pl.ds(k * CHUNK, CHUNK)]
    r = lax.broadcasted_iota(jnp.int32, (EBOXP - EBOX, CHUNK), 0)
    col = lax.broadcasted_iota(jnp.int32, (EBOXP - EBOX, CHUNK), 1)
    pad = N + lax.rem(r * CHUNK + col, N_PAD - N)
    s_ref[pl.ds(EBOX, EBOXP - EBOX), :] = pad
    d_ref[pl.ds(EBOX, EBOXP - EBOX), :] = pad


def _edge_split(edge_index):
    return pl.pallas_call(
        _edge_split_body,
        grid=(ECHUNK // EBOX,),
        in_specs=[pl.BlockSpec((2, EBOX * CHUNK), lambda i: (0, i))],
        out_specs=[pl.BlockSpec((EBOXP, CHUNK), lambda i: (i, 0)),
                   pl.BlockSpec((EBOXP, CHUNK), lambda i: (i, 0))],
        out_shape=[jax.ShapeDtypeStruct((NROW, CHUNK), jnp.int32),
                   jax.ShapeDtypeStruct((NROW, CHUNK), jnp.int32)],
    )(edge_index)


@functools.partial(
    pl.kernel,
    out_type=jax.ShapeDtypeStruct((NC, N_PAD), jnp.float32),
    mesh=_mesh,
    scratch_types=[
        pltpu.VMEM_SHARED((N_PAD,), jnp.float32),
        pltpu.VMEM((NCHUNK, CHUNK), jnp.int32),
        pltpu.VMEM((CHUNK,), jnp.float32),
        pltpu.VMEM((RPT,), jnp.float32),
    ],
    compiler_params=_sc_params,
)
def _sc_degree(dst_hbm, deg_out, deg_sh, idx_v, ones_v, zero_v):
    c = lax.axis_index("c")
    s = lax.axis_index("s")
    wid = c * NS + s
    for i in range(CHUNK // L):
        ones_v[pl.ds(i * L, L)] = jnp.full((L,), 1.0, jnp.float32)
    for i in range(RPT // L):
        zero_v[pl.ds(i * L, L)] = jnp.zeros((L,), jnp.float32)
    pltpu.sync_copy(zero_v, deg_sh.at[pl.ds(s * RPT, RPT)])
    pltpu.sync_copy(dst_hbm.at[pl.ds(wid * NCHUNK, NCHUNK)], idx_v)
    plsc.subcore_barrier()

    def body(j, carry):
        pltpu.sync_copy(ones_v, deg_sh.at[idx_v.at[j]], add=True)
        return carry

    lax.fori_loop(0, NCHUNK, body, 0)
    plsc.subcore_barrier()
    pltpu.sync_copy(deg_sh.at[pl.ds(s * RPT, RPT)],
                    deg_out.at[c, pl.ds(s * RPT, RPT)])


@functools.partial(
    pl.kernel,
    out_type=jax.ShapeDtypeStruct((N_PAD, NC * D_H), jnp.float32),
    mesh=_mesh,
    scratch_types=[
        pltpu.VMEM_SHARED((N_PAD, D_H), jnp.float32),
        pltpu.VMEM((NCHUNK, CHUNK), jnp.int32),
        pltpu.VMEM((NCHUNK, CHUNK), jnp.int32),
        pltpu.VMEM((RING, CHUNK, D_H), jnp.float32),
        pltpu.VMEM((64, D_H), jnp.float32),
        pltpu.SemaphoreType.DMA((RING,)),
        pltpu.SemaphoreType.DMA((RING,)),
    ],
    compiler_params=_sc_params,
)
def _sc_scatter(g_hbm, src_hbm, dst_hbm, acc_out,
                acc_sh, src_v, dst_v, rows, zero_v, sem_g, sem_s):
    c = lax.axis_index("c")
    s = lax.axis_index("s")
    wid = c * NS + s
    for i in range(64):
        for k in range(D_H // L):
            zero_v[i, pl.ds(k * L, L)] = jnp.zeros((L,), jnp.float32)
    for k in range(RPT // 64):
        pltpu.sync_copy(zero_v, acc_sh.at[pl.ds(s * RPT + k * 64, 64)])
    pltpu.sync_copy(src_hbm.at[pl.ds(wid * NCHUNK, NCHUNK)], src_v)
    pltpu.sync_copy(dst_hbm.at[pl.ds(wid * NCHUNK, NCHUNK)], dst_v)
    plsc.subcore_barrier()
    for k in range(RING):
        pltpu.async_copy(g_hbm.at[src_v.at[k]], rows.at[k], sem_g.at[k])

    def body(t, carry):
        base = RING * t
        for k in range(RING):
            pltpu.make_async_copy(g_hbm.at[src_v.at[base + k]],
                                  rows.at[k], sem_g.at[k]).wait()
            pltpu.async_copy(rows.at[k], acc_sh.at[dst_v.at[base + k]],
                             sem_s.at[k], add=True)
        for k in range(RING):
            pltpu.make_async_copy(rows.at[k], acc_sh.at[dst_v.at[base + k]],
                                  sem_s.at[k]).wait()

            @pl.when(base + k + RING < NCHUNK)
            def _():
                pltpu.async_copy(g_hbm.at[src_v.at[base + k + RING]],
                                 rows.at[k], sem_g.at[k])

        return carry

    lax.fori_loop(0, NCHUNK // RING, body, 0)
    plsc.subcore_barrier()
    pltpu.sync_copy(acc_sh.at[pl.ds(s * RPT, RPT)],
                    acc_out.at[pl.ds(s * RPT, RPT), pl.ds(c * D_H, D_H)])


def _dinv_col(deg_ref):
    dv = lax.rsqrt(deg_ref[0:1, :] + deg_ref[1:2, :] + 1.0)
    return jnp.transpose(dv, (1, 0))


def _mm_scale_body(x_ref, w_ref, deg_ref, g_ref):
    g_ref[...] = jnp.dot(x_ref[...] * _dinv_col(deg_ref), w_ref[...],
                         preferred_element_type=jnp.float32)


def _mm_scale(x, w, deg_part):
    m, k = x.shape
    n = w.shape[1]
    return pl.pallas_call(
        _mm_scale_body,
        grid=(m // BLK,),
        in_specs=[pl.BlockSpec((BLK, k), lambda i: (i, 0)),
                  pl.BlockSpec((k, n), lambda i: (0, 0)),
                  pl.BlockSpec((NC, BLK), lambda i: (0, i))],
        out_specs=pl.BlockSpec((BLK, n), lambda i: (i, 0)),
        out_shape=jax.ShapeDtypeStruct((m, n), jnp.float32),
    )(x, w, deg_part)


def _layer2_body(acc_ref, g_ref, deg_ref, b_ref, w_ref, o_ref):
    dv = _dinv_col(deg_ref)
    accsum = acc_ref[:, :D_H] + acc_ref[:, D_H:]
    h1 = jnp.maximum(dv * (accsum + g_ref[...]) + b_ref[...], 0.0)
    o_ref[...] = jnp.dot(h1, w_ref[...], preferred_element_type=jnp.float32) * dv


def _layer2(acc, g1, deg_part, b1, w2):
    return pl.pallas_call(
        _layer2_body,
        grid=(N_PAD // BLK,),
        in_specs=[pl.BlockSpec((BLK, NC * D_H), lambda i: (i, 0)),
                  pl.BlockSpec((BLK, D_H), lambda i: (i, 0)),
                  pl.BlockSpec((NC, BLK), lambda i: (0, i)),
                  pl.BlockSpec((1, D_H), lambda i: (0, 0)),
                  pl.BlockSpec((D_H, D_H), lambda i: (0, 0))],
        out_specs=pl.BlockSpec((BLK, D_H), lambda i: (i, 0)),
        out_shape=jax.ShapeDtypeStruct((N_PAD, D_H), jnp.float32),
    )(acc, g1, deg_part, b1, w2)


def _pool_body(acc_ref, g_ref, deg_ref, b_ref, batch_ref, wl_ref, bl_ref,
               o_ref, sums_scr, cnt_scr):
    i = pl.program_id(0)

    @pl.when(i == 0)
    def _():
        sums_scr[...] = jnp.zeros_like(sums_scr)
        cnt_scr[...] = jnp.zeros_like(cnt_scr)

    dv = _dinv_col(deg_ref)
    accsum = acc_ref[:, :D_H] + acc_ref[:, D_H:]
    h2 = jnp.maximum(dv * (accsum + g_ref[...]) + b_ref[...], 0.0)
    pt = (batch_ref[...] == lax.broadcasted_iota(jnp.int32, (NG, BLK), 0))
    pt = pt.astype(jnp.float32)
    sums_scr[...] += lax.dot_general(pt, h2, (((1,), (0,)), ((), ())),
                                     preferred_element_type=jnp.float32)
    cnt_scr[...] += lax.dot_general(pt, jnp.ones((BLK, 1), jnp.float32),
                                    (((1,), (0,)), ((), ())),
                                    preferred_element_type=jnp.float32)

    @pl.when(i == pl.num_programs(0) - 1)
    def _():
        pooled = sums_scr[...] / jnp.maximum(cnt_scr[...], 1.0)
        o_ref[...] = jnp.dot(pooled, wl_ref[...],
                             preferred_element_type=jnp.float32) + bl_ref[...]


def _pool(acc, g2, deg_part, b2, batch_row, wlin, blin):
    return pl.pallas_call(
        _pool_body,
        grid=(N_PAD // BLK,),
        in_specs=[pl.BlockSpec((BLK, NC * D_H), lambda i: (i, 0)),
                  pl.BlockSpec((BLK, D_H), lambda i: (i, 0)),
                  pl.BlockSpec((NC, BLK), lambda i: (0, i)),
                  pl.BlockSpec((1, D_H), lambda i: (0, 0)),
                  pl.BlockSpec((1, BLK), lambda i: (0, i)),
                  pl.BlockSpec((D_H, 2), lambda i: (0, 0)),
                  pl.BlockSpec((1, 2), lambda i: (0, 0))],
        out_specs=pl.BlockSpec((NG, 2), lambda i: (0, 0)),
        out_shape=jax.ShapeDtypeStruct((NG, 2), jnp.float32),
        scratch_shapes=[pltpu.VMEM((NG, D_H), jnp.float32),
                        pltpu.VMEM((NG, 1), jnp.float32)],
    )(acc, g2, deg_part, b2, batch_row, wlin, blin)


def kernel(x, edge_index, batch, W1, b1, W2, b2, Wlin, blin):
    src2d, dst2d = _edge_split(edge_index.astype(jnp.int32))
    x_pad = jnp.concatenate([x, jnp.zeros((N_PAD - N, D_IN), x.dtype)])
    batch_row = jnp.concatenate(
        [batch.astype(jnp.int32), jnp.full((N_PAD - N,), NG, jnp.int32)]
    ).reshape(1, N_PAD)

    deg_part = _sc_degree(dst2d)
    g1 = _mm_scale(x_pad, W1, deg_part)
    acc1 = _sc_scatter(g1, src2d, dst2d)
    g2 = _layer2(acc1, g1, deg_part, b1.reshape(1, D_H), W2)
    acc2 = _sc_scatter(g2, src2d, dst2d)
    return _pool(acc2, g2, deg_part, b2.reshape(1, D_H), batch_row,
                 Wlin, blin.reshape(1, 2))


# speedup vs baseline: 1.0615x; 1.0031x over previous
import functools

import jax
import jax.numpy as jnp
from jax import lax
from jax.experimental import pallas as pl
from jax.experimental.pallas import tpu as pltpu
from jax.experimental.pallas import tpu_sc as plsc

N = 10000
E = 320000
D_IN = 128
D_H = 64
NG = 64

NC = 2
NS = 16
L = 16
NW = NC * NS
CHUNK = 128
NCHUNK = 80
ECHUNK = E // CHUNK
EBOX = 500
EBOXP = 512
NROW = NW * NCHUNK
N_PAD = 10240
RPT = N_PAD // NS
RING = 8
BLK = 1024

_mesh = plsc.VectorSubcoreMesh(core_axis_name="c", subcore_axis_name="s")
_sc_params = pltpu.CompilerParams(use_tc_tiling_on_sc=False)


def _edge_split_body(e_ref, s_ref, d_ref):
    for k in range(EBOX):
        s_ref[pl.ds(k, 1), :] = e_ref[0:1, pl.ds(k * CHUNK, CHUNK)]
        d_ref[pl.ds(k, 1), :] = e_ref[1:2, pl.ds(k * CHUNK, CHUNK)]
    r = lax.broadcasted_iota(jnp.int32, (EBOXP - EBOX, CHUNK), 0)
    col = lax.broadcasted_iota(jnp.int32, (EBOXP - EBOX, CHUNK), 1)
    pad = N + lax.rem(r * CHUNK + col, N_PAD - N)
    s_ref[pl.ds(EBOX, EBOXP - EBOX), :] = pad
    d_ref[pl.ds(EBOX, EBOXP - EBOX), :] = pad


def _edge_split(edge_index):
    return pl.pallas_call(
        _edge_split_body,
        grid=(ECHUNK // EBOX,),
        in_specs=[pl.BlockSpec((2, EBOX * CHUNK), lambda i: (0, i))],
        out_specs=[pl.BlockSpec((EBOXP, CHUNK), lambda i: (i, 0)),
                   pl.BlockSpec((EBOXP, CHUNK), lambda i: (i, 0))],
        out_shape=[jax.ShapeDtypeStruct((NROW, CHUNK), jnp.int32),
                   jax.ShapeDtypeStruct((NROW, CHUNK), jnp.int32)],
    )(edge_index)


@functools.partial(
    pl.kernel,
    out_type=jax.ShapeDtypeStruct((NC, N_PAD), jnp.float32),
    mesh=_mesh,
    scratch_types=[
        pltpu.VMEM_SHARED((N_PAD,), jnp.float32),
        pltpu.VMEM((NCHUNK, CHUNK), jnp.int32),
        pltpu.VMEM((CHUNK,), jnp.float32),
        pltpu.VMEM((RPT,), jnp.float32),
    ],
    compiler_params=_sc_params,
)
def _sc_degree(dst_hbm, deg_out, deg_sh, idx_v, ones_v, zero_v):
    c = lax.axis_index("c")
    s = lax.axis_index("s")
    wid = c * NS + s
    for i in range(CHUNK // L):
        ones_v[pl.ds(i * L, L)] = jnp.full((L,), 1.0, jnp.float32)
    for i in range(RPT // L):
        zero_v[pl.ds(i * L, L)] = jnp.zeros((L,), jnp.float32)
    pltpu.sync_copy(zero_v, deg_sh.at[pl.ds(s * RPT, RPT)])
    pltpu.sync_copy(dst_hbm.at[pl.ds(wid * NCHUNK, NCHUNK)], idx_v)
    plsc.subcore_barrier()

    def body(j, carry):
        pltpu.sync_copy(ones_v, deg_sh.at[idx_v.at[j]], add=True)
        return carry

    lax.fori_loop(0, NCHUNK, body, 0)
    plsc.subcore_barrier()
    pltpu.sync_copy(deg_sh.at[pl.ds(s * RPT, RPT)],
                    deg_out.at[c, pl.ds(s * RPT, RPT)])


@functools.partial(
    pl.kernel,
    out_type=jax.ShapeDtypeStruct((N_PAD, NC * D_H), jnp.float32),
    mesh=_mesh,
    scratch_types=[
        pltpu.VMEM_SHARED((N_PAD, D_H), jnp.float32),
        pltpu.VMEM((NCHUNK, CHUNK), jnp.int32),
        pltpu.VMEM((NCHUNK, CHUNK), jnp.int32),
        pltpu.VMEM((RING, CHUNK, D_H), jnp.float32),
        pltpu.VMEM((64, D_H), jnp.float32),
        pltpu.SemaphoreType.DMA((RING,)),
        pltpu.SemaphoreType.DMA((RING,)),
    ],
    compiler_params=_sc_params,
)
def _sc_scatter(g_hbm, src_hbm, dst_hbm, acc_out,
                acc_sh, src_v, dst_v, rows, zero_v, sem_g, sem_s):
    c = lax.axis_index("c")
    s = lax.axis_index("s")
    wid = c * NS + s
    for i in range(64):
        for k in range(D_H // L):
            zero_v[i, pl.ds(k * L, L)] = jnp.zeros((L,), jnp.float32)
    for k in range(RPT // 64):
        pltpu.sync_copy(zero_v, acc_sh.at[pl.ds(s * RPT + k * 64, 64)])
    pltpu.sync_copy(src_hbm.at[pl.ds(wid * NCHUNK, NCHUNK)], src_v)
    pltpu.sync_copy(dst_hbm.at[pl.ds(wid * NCHUNK, NCHUNK)], dst_v)
    plsc.subcore_barrier()
    for k in range(RING):
        pltpu.async_copy(g_hbm.at[src_v.at[k]], rows.at[k], sem_g.at[k])

    def body(t, carry):
        base = RING * t
        for k in range(RING):
            pltpu.make_async_copy(g_hbm.at[src_v.at[base + k]],
                                  rows.at[k], sem_g.at[k]).wait()
            pltpu.async_copy(rows.at[k], acc_sh.at[dst_v.at[base + k]],
                             sem_s.at[k], add=True)
        for k in range(RING):
            pltpu.make_async_copy(rows.at[k], acc_sh.at[dst_v.at[base + k]],
                                  sem_s.at[k]).wait()

            @pl.when(base + k + RING < NCHUNK)
            def _():
                pltpu.async_copy(g_hbm.at[src_v.at[base + k + RING]],
                                 rows.at[k], sem_g.at[k])

        return carry

    lax.fori_loop(0, NCHUNK // RING, body, 0)
    plsc.subcore_barrier()
    pltpu.sync_copy(acc_sh.at[pl.ds(s * RPT, RPT)],
                    acc_out.at[pl.ds(s * RPT, RPT), pl.ds(c * D_H, D_H)])


def _dinv_col(deg_ref):
    dv = lax.rsqrt(deg_ref[0:1, :] + deg_ref[1:2, :] + 1.0)
    return jnp.transpose(dv, (1, 0))


def _mm_scale_body(x_ref, w_ref, deg_ref, g_ref):
    g_ref[...] = jnp.dot(x_ref[...], w_ref[...],
                         preferred_element_type=jnp.float32) * _dinv_col(deg_ref)


def _mm_scale(x, w, deg_part):
    m, k = x.shape
    n = w.shape[1]
    return pl.pallas_call(
        _mm_scale_body,
        grid=(m // BLK,),
        in_specs=[pl.BlockSpec((BLK, k), lambda i: (i, 0)),
                  pl.BlockSpec((k, n), lambda i: (0, 0)),
                  pl.BlockSpec((NC, BLK), lambda i: (0, i))],
        out_specs=pl.BlockSpec((BLK, n), lambda i: (i, 0)),
        out_shape=jax.ShapeDtypeStruct((m, n), jnp.float32),
    )(x, w, deg_part)


def _layer2_body(acc_ref, g_ref, deg_ref, b_ref, w_ref, o_ref):
    dv = _dinv_col(deg_ref)
    accsum = acc_ref[:, :D_H] + acc_ref[:, D_H:]
    h1 = jnp.maximum(dv * (accsum + g_ref[...]) + b_ref[...], 0.0)
    o_ref[...] = jnp.dot(h1, w_ref[...], preferred_element_type=jnp.float32) * dv


def _layer2(acc, g1, deg_part, b1, w2):
    return pl.pallas_call(
        _layer2_body,
        grid=(N_PAD // BLK,),
        in_specs=[pl.BlockSpec((BLK, NC * D_H), lambda i: (i, 0)),
                  pl.BlockSpec((BLK, D_H), lambda i: (i, 0)),
                  pl.BlockSpec((NC, BLK), lambda i: (0, i)),
                  pl.BlockSpec((1, D_H), lambda i: (0, 0)),
                  pl.BlockSpec((D_H, D_H), lambda i: (0, 0))],
        out_specs=pl.BlockSpec((BLK, D_H), lambda i: (i, 0)),
        out_shape=jax.ShapeDtypeStruct((N_PAD, D_H), jnp.float32),
    )(acc, g1, deg_part, b1, w2)


def _pool_body(acc_ref, g_ref, deg_ref, b_ref, batch_ref, wl_ref, bl_ref,
               o_ref, sums_scr, cnt_scr):
    i = pl.program_id(0)

    @pl.when(i == 0)
    def _():
        sums_scr[...] = jnp.zeros_like(sums_scr)
        cnt_scr[...] = jnp.zeros_like(cnt_scr)

    dv = _dinv_col(deg_ref)
    accsum = acc_ref[:, :D_H] + acc_ref[:, D_H:]
    h2 = jnp.maximum(dv * (accsum + g_ref[...]) + b_ref[...], 0.0)
    pt = (batch_ref[...] == lax.broadcasted_iota(jnp.int32, (NG, BLK), 0))
    pt = pt.astype(jnp.float32)
    sums_scr[...] += lax.dot_general(pt, h2, (((1,), (0,)), ((), ())),
                                     preferred_element_type=jnp.float32)
    cnt_scr[...] += lax.dot_general(pt, jnp.ones((BLK, 1), jnp.float32),
                                    (((1,), (0,)), ((), ())),
                                    preferred_element_type=jnp.float32)

    @pl.when(i == pl.num_programs(0) - 1)
    def _():
        pooled = sums_scr[...] / jnp.maximum(cnt_scr[...], 1.0)
        o_ref[...] = jnp.dot(pooled, wl_ref[...],
                             preferred_element_type=jnp.float32) + bl_ref[...]


def _pool(acc, g2, deg_part, b2, batch_row, wlin, blin):
    return pl.pallas_call(
        _pool_body,
        grid=(N_PAD // BLK,),
        in_specs=[pl.BlockSpec((BLK, NC * D_H), lambda i: (i, 0)),
                  pl.BlockSpec((BLK, D_H), lambda i: (i, 0)),
                  pl.BlockSpec((NC, BLK), lambda i: (0, i)),
                  pl.BlockSpec((1, D_H), lambda i: (0, 0)),
                  pl.BlockSpec((1, BLK), lambda i: (0, i)),
                  pl.BlockSpec((D_H, 2), lambda i: (0, 0)),
                  pl.BlockSpec((1, 2), lambda i: (0, 0))],
        out_specs=pl.BlockSpec((NG, 2), lambda i: (0, 0)),
        out_shape=jax.ShapeDtypeStruct((NG, 2), jnp.float32),
        scratch_shapes=[pltpu.VMEM((NG, D_H), jnp.float32),
                        pltpu.VMEM((NG, 1), jnp.float32)],
    )(acc, g2, deg_part, b2, batch_row, wlin, blin)


def kernel(x, edge_index, batch, W1, b1, W2, b2, Wlin, blin):
    src2d, dst2d = _edge_split(edge_index.astype(jnp.int32))
    x_pad = jnp.concatenate([x, jnp.zeros((N_PAD - N, D_IN), x.dtype)])
    batch_row = jnp.concatenate(
        [batch.astype(jnp.int32), jnp.full((N_PAD - N,), NG, jnp.int32)]
    ).reshape(1, N_PAD)

    deg_part = _sc_degree(dst2d)
    g1 = _mm_scale(x_pad, W1, deg_part)
    acc1 = _sc_scatter(g1, src2d, dst2d)
    g2 = _layer2(acc1, g1, deg_part, b1.reshape(1, D_H), W2)
    acc2 = _sc_scatter(g2, src2d, dst2d)
    return _pool(acc2, g2, deg_part, b2.reshape(1, D_H), batch_row,
                 Wlin, blin.reshape(1, 2))
